# baseline (speedup 1.0000x reference)
import functools

import jax
import jax.numpy as jnp
from jax import lax
from jax.experimental import pallas as pl
from jax.experimental.pallas import tpu as pltpu
from jax.experimental.pallas import tpu_sc as plsc

BATCH = 16384
HIST = 50
EMBED_DIM = 64

NUM_CORES = 2
NUM_SUBCORES = 16
NW = NUM_CORES * NUM_SUBCORES
CHUNK = 128
B_TOTAL = BATCH * HIST
BLOCKS = B_TOTAL // (NW * CHUNK)
BPS = 2
NSUPER = BLOCKS // BPS
FT = EMBED_DIM // 8

_mesh = plsc.VectorSubcoreMesh(core_axis_name="c", subcore_axis_name="s")


@functools.partial(
    pl.kernel,
    mesh=_mesh,
    compiler_params=pltpu.CompilerParams(
        use_tc_tiling_on_sc=False, needs_layout_passes=False
    ),
    out_type=jax.ShapeDtypeStruct((HIST, FT, BATCH // CHUNK, 8, CHUNK), jnp.float32),
    scratch_types=[
        pltpu.VMEM((BLOCKS, CHUNK), jnp.int32),
        pltpu.VMEM((2, BPS * CHUNK, EMBED_DIM), jnp.float32),
        pltpu.VMEM((2, BPS, FT, 8, CHUNK), jnp.float32),
        pltpu.SemaphoreType.DMA,
        pltpu.SemaphoreType.DMA,
    ],
)
def _gather_kernel(table_hbm, idx_hbm, out_hbm, idx_v, rows_v, trans_v, gsem, wsem):
    wid = lax.axis_index("s") * NUM_CORES + lax.axis_index("c")
    pltpu.sync_copy(idx_hbm.at[wid], idx_v)

    lanes = lax.iota(jnp.int32, 16)

    def start_bank(t, bank):
        for u in range(BPS):
            pltpu.async_copy(
                table_hbm.at[idx_v.at[BPS * t + u]],
                rows_v.at[bank, pl.ds(u * CHUNK, CHUNK)],
                gsem,
            )

    start_bank(0, 0)

    def step(t, _):
        cur = t % 2

        @pl.when(t < NSUPER - 1)
        def _():
            start_bank(t + 1, 1 - cur)

        pltpu.make_async_copy(
            table_hbm.at[idx_v.at[0]], rows_v.at[cur], gsem
        ).wait()

        @pl.when(t >= 2)
        def _():
            for blk in range(BPS):
                pltpu.make_async_copy(
                    out_hbm.at[0, pl.ds(0, 8), 0], trans_v.at[cur, blk], wsem
                ).wait()

        for blk in range(BPS):
            for fr in range(FT):
                for r in range(8):
                    col = jnp.full((16,), 8 * fr + r, jnp.int32)
                    for cb in range(CHUNK // 16):
                        ridx = blk * CHUNK + cb * 16 + lanes
                        val = plsc.load_gather(rows_v.at[cur], [ridx, col])
                        trans_v[cur, blk, fr, r, pl.ds(cb * 16, 16)] = val
            f = wid * BLOCKS + BPS * t + blk
            h = f // (BATCH // CHUNK)
            bt = f % (BATCH // CHUNK)
            pltpu.async_copy(
                trans_v.at[cur, blk], out_hbm.at[h, pl.ds(0, FT), bt], wsem
            )

        return ()

    lax.fori_loop(0, NSUPER, step, ())
    for bank in range(2):
        for blk in range(BPS):
            pltpu.make_async_copy(
                out_hbm.at[0, pl.ds(0, 8), 0], trans_v.at[bank, blk], wsem
            ).wait()


def kernel(x, table):
    xf = x.T.reshape(NW, BLOCKS, CHUNK).astype(jnp.int32)
    out5 = _gather_kernel(table, xf)
    return out5.transpose(2, 4, 0, 1, 3).reshape(BATCH, HIST, EMBED_DIM)


# speedup vs baseline: 1.4630x; 1.4630x over previous
import functools

import jax
import jax.numpy as jnp
from jax import lax
from jax.experimental import pallas as pl
from jax.experimental.pallas import tpu as pltpu
from jax.experimental.pallas import tpu_sc as plsc

BATCH = 16384
HIST = 50
EMBED_DIM = 64

NUM_CORES = 2
NUM_SUBCORES = 16
NW = NUM_CORES * NUM_SUBCORES
CHUNK = 128
B_TOTAL = BATCH * HIST
STEPS = B_TOTAL // (NW * CHUNK)
GPB = 2
BANK_ROWS = GPB * CHUNK
NBANK = 4
NSUPER = STEPS // GPB
NGROUP = NSUPER // NBANK

_mesh = plsc.VectorSubcoreMesh(core_axis_name="c", subcore_axis_name="s")


@functools.partial(
    pl.kernel,
    mesh=_mesh,
    compiler_params=pltpu.CompilerParams(use_tc_tiling_on_sc=False),
    out_type=jax.ShapeDtypeStruct((NW, NSUPER, BANK_ROWS, EMBED_DIM), jnp.float32),
    scratch_types=[
        pltpu.VMEM((STEPS, CHUNK), jnp.int32),
        pltpu.VMEM((NBANK, BANK_ROWS, EMBED_DIM), jnp.float32),
        [pltpu.SemaphoreType.DMA] * NBANK,
        [pltpu.SemaphoreType.DMA] * NBANK,
    ],
)
def _gather_kernel(table_hbm, idx_hbm, out_hbm, idx_v, rows_v, gsems, wsems):
    wid = lax.axis_index("s") * NUM_CORES + lax.axis_index("c")
    pltpu.sync_copy(idx_hbm.at[wid], idx_v)

    def start_bank(t, bank):
        for u in range(GPB):
            pltpu.async_copy(
                table_hbm.at[idx_v.at[GPB * t + u]],
                rows_v.at[bank, pl.ds(u * CHUNK, CHUNK)],
                gsems[bank],
            )

    def wait_gathers(bank):
        pltpu.make_async_copy(
            table_hbm.at[idx_v.at[0]], rows_v.at[bank], gsems[bank]
        ).wait()

    def wait_write(bank):
        pltpu.make_async_copy(
            out_hbm.at[0, 0], rows_v.at[bank], wsems[bank]
        ).wait()

    start_bank(0, 0)
    start_bank(1, 1)

    def group(g, _):
        for u in range(NBANK):
            t = NBANK * g + u
            b2 = (u + 2) % NBANK

            @pl.when(t >= 2)
            def _():
                wait_write(b2)

            @pl.when(t + 2 < NSUPER)
            def _():
                start_bank(t + 2, b2)

            wait_gathers(u)
            pltpu.async_copy(rows_v.at[u], out_hbm.at[wid, t], wsems[u])
        return ()

    lax.fori_loop(0, NGROUP, group, ())
    wait_write(2)
    wait_write(3)


def kernel(x, table):
    xf = x.reshape(NW, STEPS, CHUNK).astype(jnp.int32)
    out = _gather_kernel(table, xf)
    return out.reshape(BATCH, HIST, EMBED_DIM)


# speedup vs baseline: 1.4643x; 1.0009x over previous
import functools

import jax
import jax.numpy as jnp
from jax import lax
from jax.experimental import pallas as pl
from jax.experimental.pallas import tpu as pltpu
from jax.experimental.pallas import tpu_sc as plsc

BATCH = 16384
HIST = 50
EMBED_DIM = 64

NUM_CORES = 2
NUM_SUBCORES = 16
NW = NUM_CORES * NUM_SUBCORES
CHUNK = 128
B_TOTAL = BATCH * HIST
BLOCKS = B_TOTAL // (NW * CHUNK)
BPS = 2
NSUPER = BLOCKS // BPS
FT = EMBED_DIM // 8
PITCH = EMBED_DIM
BT = BATCH // CHUNK

_mesh = plsc.VectorSubcoreMesh(core_axis_name="c", subcore_axis_name="s")


@functools.partial(
    pl.kernel,
    mesh=_mesh,
    compiler_params=pltpu.CompilerParams(
        use_tc_tiling_on_sc=False, needs_layout_passes=False
    ),
    out_type=jax.ShapeDtypeStruct((HIST, FT, BT, 8, CHUNK), jnp.float32),
    scratch_types=[
        pltpu.VMEM((BLOCKS, CHUNK), jnp.int32),
        pltpu.VMEM((2, BPS * CHUNK, PITCH), jnp.float32),
        pltpu.VMEM((2, BPS, FT, 8, CHUNK), jnp.float32),
        [pltpu.SemaphoreType.DMA] * 2,
        [pltpu.SemaphoreType.DMA] * 2,
    ],
)
def _gather_kernel(table_hbm, idx_hbm, out_hbm, idx_v, rows_v, trans_v, gsems, wsems):
    wid = lax.axis_index("s") * NUM_CORES + lax.axis_index("c")
    pltpu.sync_copy(idx_hbm.at[wid], idx_v)

    lanes = lax.iota(jnp.int32, 16)

    def start_bank(t, bank):
        for u in range(BPS):
            pltpu.async_copy(
                table_hbm.at[idx_v.at[BPS * t + u]],
                rows_v.at[bank, pl.ds(u * CHUNK, CHUNK), pl.ds(0, EMBED_DIM)],
                gsems[bank],
            )

    def wait_gathers(bank):
        pltpu.make_async_copy(
            table_hbm.at[idx_v.at[0]],
            rows_v.at[bank, pl.ds(0, BPS * CHUNK), pl.ds(0, EMBED_DIM)],
            gsems[bank],
        ).wait()

    def wait_writes(bank):
        for blk in range(BPS):
            pltpu.make_async_copy(
                out_hbm.at[0, pl.ds(0, FT), 0], trans_v.at[bank, blk], wsems[bank]
            ).wait()

    start_bank(0, 0)

    def group(g, _):
        for b in range(2):
            t = 2 * g + b

            @pl.when(t + 1 < NSUPER)
            def _():
                start_bank(t + 1, 1 - b)

            wait_gathers(b)

            @pl.when(t >= 2)
            def _():
                wait_writes(b)

            for blk in range(BPS):
                @plsc.parallel_loop(0, EMBED_DIM, unroll=4)
                def _(i, _blk=blk, _b=b):
                    col = jnp.full((16,), i, jnp.int32)
                    fr = i // 8
                    r = i - 8 * fr
                    for cb in range(CHUNK // 16):
                        ridx = _blk * CHUNK + cb * 16 + lanes
                        val = plsc.load_gather(rows_v.at[_b], [ridx, col])
                        trans_v[_b, _blk, fr, r, pl.ds(cb * 16, 16)] = val
                f = wid * BLOCKS + BPS * t + blk
                h = f // BT
                bt = f % BT
                pltpu.async_copy(
                    trans_v.at[b, blk], out_hbm.at[h, pl.ds(0, FT), bt], wsems[b]
                )
        return ()

    lax.fori_loop(0, NSUPER // 2, group, ())
    wait_writes(0)
    wait_writes(1)


def kernel(x, table):
    xf = x.T.reshape(NW, BLOCKS, CHUNK).astype(jnp.int32)
    out5 = _gather_kernel(table, xf)
    return out5.transpose(2, 4, 0, 1, 3).reshape(BATCH, HIST, EMBED_DIM)


# speedup vs baseline: 2.0369x; 1.3910x over previous
import jax
import jax.numpy as jnp
from jax import lax
from jax.experimental import pallas as pl
from jax.experimental.pallas import tpu as pltpu
from jax.experimental.pallas import tpu_sc as plsc

import functools

BATCH = 16384
HIST = 50
EMBED_DIM = 64

NUM_CORES = 2
NUM_SUBCORES = 16
NW = NUM_CORES * NUM_SUBCORES
CHUNK = 128
B_TOTAL = BATCH * HIST
BLOCKS = B_TOTAL // (NW * CHUNK)
BPS = 2
NSUPER = BLOCKS // BPS
FT = EMBED_DIM // 8
BT = BATCH // CHUNK

_mesh = plsc.VectorSubcoreMesh(core_axis_name="c", subcore_axis_name="s")


@functools.partial(
    pl.kernel,
    mesh=_mesh,
    compiler_params=pltpu.CompilerParams(
        use_tc_tiling_on_sc=False, needs_layout_passes=False
    ),
    out_type=jax.ShapeDtypeStruct((HIST, FT, BT, 8 * CHUNK), jnp.float32),
    scratch_types=[
        pltpu.VMEM((BLOCKS, CHUNK), jnp.int32),
        pltpu.VMEM((2, BPS * CHUNK, EMBED_DIM), jnp.float32),
        pltpu.VMEM((2, BPS, FT, 8 * CHUNK), jnp.float32),
        [pltpu.SemaphoreType.DMA] * 2,
        [pltpu.SemaphoreType.DMA] * 2,
    ],
)
def _gather_kernel(table_hbm, idx_hbm, out_hbm, idx_v, rows_v, trans_v, gsems, wsems):
    wid = lax.axis_index("s") * NUM_CORES + lax.axis_index("c")
    pltpu.sync_copy(idx_hbm.at[wid], idx_v)

    lanes = lax.iota(jnp.int32, 16)
    perm = [(lanes + d) & 15 for d in range(16)]

    def start_bank(t, bank):
        for u in range(BPS):
            pltpu.async_copy(
                table_hbm.at[idx_v.at[BPS * t + u]],
                rows_v.at[bank, pl.ds(u * CHUNK, CHUNK)],
                gsems[bank],
            )

    def wait_gathers(bank):
        pltpu.make_async_copy(
            table_hbm.at[idx_v.at[0]], rows_v.at[bank], gsems[bank]
        ).wait()

    def wait_writes(bank):
        for blk in range(BPS):
            pltpu.make_async_copy(
                out_hbm.at[0, pl.ds(0, FT), 0], trans_v.at[bank, blk], wsems[bank]
            ).wait()

    start_bank(0, 0)

    def group(g, _):
        for b in range(2):
            t = 2 * g + b

            @pl.when(t + 1 < NSUPER)
            def _():
                start_bank(t + 1, 1 - b)

            wait_gathers(b)

            @pl.when(t >= 2)
            def _():
                wait_writes(b)

            for blk in range(BPS):
                @plsc.parallel_loop(0, 32, unroll=2)
                def _(i, _blk=blk, _b=b):
                    f0 = (i // 8) * 16
                    c0 = (i % 8) * 16
                    ridx = _blk * CHUNK + c0 + lanes
                    for d in range(16):
                        fvec = f0 + perm[d]
                        val = plsc.load_gather(rows_v.at[_b], [ridx, fvec])
                        fr_vec = fvec // 8
                        rc_vec = (fvec & 7) * CHUNK + c0 + lanes
                        plsc.store_scatter(
                            trans_v.at[_b, _blk], [fr_vec, rc_vec], val
                        )

                f = wid * BLOCKS + BPS * t + blk
                h = f // BT
                bt = f % BT
                pltpu.async_copy(
                    trans_v.at[b, blk], out_hbm.at[h, pl.ds(0, FT), bt], wsems[b]
                )
        return ()

    lax.fori_loop(0, NSUPER // 2, group, ())
    wait_writes(0)
    wait_writes(1)


def kernel(x, table):
    xf = x.T.reshape(NW, BLOCKS, CHUNK).astype(jnp.int32)
    out4 = _gather_kernel(table, xf)
    out5 = out4.reshape(HIST, FT, BT, 8, CHUNK)
    return out5.transpose(2, 4, 0, 1, 3).reshape(BATCH, HIST, EMBED_DIM)


# speedup vs baseline: 2.2484x; 1.1039x over previous
import jax
import jax.numpy as jnp
from jax import lax
from jax.experimental import pallas as pl
from jax.experimental.pallas import tpu as pltpu
from jax.experimental.pallas import tpu_sc as plsc

import functools

BATCH = 16384
HIST = 50
EMBED_DIM = 64

NUM_CORES = 2
NUM_SUBCORES = 16
NW = NUM_CORES * NUM_SUBCORES
CHUNK = 128
B_TOTAL = BATCH * HIST
BLOCKS = B_TOTAL // (NW * CHUNK)
BPS = 2
NSUPER = BLOCKS // BPS
FT = EMBED_DIM // 8
BT = BATCH // CHUNK

_mesh = plsc.VectorSubcoreMesh(core_axis_name="c", subcore_axis_name="s")


@functools.partial(
    pl.kernel,
    mesh=_mesh,
    compiler_params=pltpu.CompilerParams(
        use_tc_tiling_on_sc=False, needs_layout_passes=False
    ),
    out_type=jax.ShapeDtypeStruct((HIST, FT, BT, 8 * CHUNK), jnp.float32),
    scratch_types=[
        pltpu.VMEM((BLOCKS, CHUNK), jnp.int32),
        pltpu.VMEM((2, BPS * CHUNK, EMBED_DIM), jnp.float32),
        pltpu.VMEM((2, BPS, FT, 8 * CHUNK), jnp.float32),
        [pltpu.SemaphoreType.DMA] * 2,
        [pltpu.SemaphoreType.DMA] * 2,
    ],
)
def _gather_kernel(table_hbm, idx_hbm, out_hbm, idx_v, rows_v, trans_v, gsems, wsems):
    wid = lax.axis_index("s") * NUM_CORES + lax.axis_index("c")
    pltpu.sync_copy(idx_hbm.at[wid], idx_v)

    lanes = lax.iota(jnp.int32, 16)
    perm = [(lanes + d) & 15 for d in range(16)]

    def start_bank(t, bank):
        for u in range(BPS):
            pltpu.async_copy(
                table_hbm.at[idx_v.at[BPS * t + u]],
                rows_v.at[bank, pl.ds(u * CHUNK, CHUNK)],
                gsems[bank],
            )

    def wait_gathers(bank):
        pltpu.make_async_copy(
            table_hbm.at[idx_v.at[0]], rows_v.at[bank], gsems[bank]
        ).wait()

    def wait_writes(bank):
        for blk in range(BPS):
            pltpu.make_async_copy(
                out_hbm.at[0, pl.ds(0, FT), 0], trans_v.at[bank, blk], wsems[bank]
            ).wait()

    start_bank(0, 0)

    def group(g, _):
        for b in range(2):
            t = 2 * g + b

            @pl.when(t + 1 < NSUPER)
            def _():
                start_bank(t + 1, 1 - b)

            wait_gathers(b)

            @pl.when(t >= 2)
            def _():
                wait_writes(b)

            for blk in range(BPS):
                @plsc.parallel_loop(0, 32, unroll=4)
                def _(i, _blk=blk, _b=b):
                    f0 = (i // 8) * 16
                    c0 = (i % 8) * 16
                    ridx = _blk * CHUNK + c0 + lanes
                    for d in range(16):
                        fvec = f0 + perm[d]
                        val = plsc.load_gather(rows_v.at[_b], [ridx, fvec])
                        fr_vec = fvec // 8
                        rc_vec = (fvec & 7) * CHUNK + c0 + lanes
                        plsc.store_scatter(
                            trans_v.at[_b, _blk], [fr_vec, rc_vec], val
                        )

                f = wid * BLOCKS + BPS * t + blk
                h = f // BT
                bt = f % BT
                pltpu.async_copy(
                    trans_v.at[b, blk], out_hbm.at[h, pl.ds(0, FT), bt], wsems[b]
                )
        return ()

    lax.fori_loop(0, NSUPER // 2, group, ())
    wait_writes(0)
    wait_writes(1)


def kernel(x, table):
    xf = x.T.reshape(NW, BLOCKS, CHUNK).astype(jnp.int32)
    out4 = _gather_kernel(table, xf)
    out5 = out4.reshape(HIST, FT, BT, 8, CHUNK)
    return out5.transpose(2, 4, 0, 1, 3).reshape(BATCH, HIST, EMBED_DIM)
